# baseline (speedup 1.0000x reference)
import dataclasses
import functools

import jax
import jax.numpy as jnp
from jax import lax
from jax.experimental import pallas as pl
from jax.experimental.pallas import tpu as pltpu
from jax.experimental.pallas import tpu_sc as plsc

_HIDDEN = 64
_GAMMA = 12.0
_LANES = 16
_NUM_CORES = 2
_NUM_SUBCORES = 16
_NUM_WORKERS = _NUM_CORES * _NUM_SUBCORES
_IDX_COLS = 128
_HALF = 256


@functools.partial(jax.jit, static_argnames=("batch",))
def _score(batch, hp_idx, rp_idx, tp_idx, hh_idx, rh_idx, th_idx,
           ent_p, rel_p):
    bpw = batch // _NUM_WORKERS
    rows_pw = bpw // _IDX_COLS
    n_half = bpw // _HALF
    rows_ph = _HALF // _IDX_COLS
    groups = _HALF // _LANES
    mesh = plsc.VectorSubcoreMesh(core_axis_name="c", subcore_axis_name="s")
    cp = pltpu.CompilerParams()
    if "needs_layout_passes" in pltpu.CompilerParams.__dataclass_fields__:
        cp = dataclasses.replace(cp, needs_layout_passes=False)

    @functools.partial(
        pl.kernel,
        out_type=jax.ShapeDtypeStruct((batch,), jnp.float32),
        mesh=mesh,
        compiler_params=cp,
        scratch_types=[
            pltpu.VMEM((rows_pw, _IDX_COLS), jnp.int32),
            pltpu.VMEM((rows_pw, _IDX_COLS), jnp.int32),
            pltpu.VMEM((rows_pw, _IDX_COLS), jnp.int32),
            pltpu.VMEM((rows_pw, _IDX_COLS), jnp.int32),
            pltpu.VMEM((rows_pw, _IDX_COLS), jnp.int32),
            pltpu.VMEM((rows_pw, _IDX_COLS), jnp.int32),
            pltpu.VMEM((_HALF, 2 * _HIDDEN), jnp.float32),
            pltpu.VMEM((_HALF, 2 * _HIDDEN), jnp.float32),
            pltpu.VMEM((_HALF, 2 * _HIDDEN), jnp.float32),
            pltpu.VMEM((bpw,), jnp.float32),
            pltpu.SemaphoreType.DMA,
        ],
    )
    def k(ent_hbm, rel_hbm, hp_hbm, rp_hbm, tp_hbm, hh_hbm, rh_hbm, th_hbm,
          out_hbm, hp_v, rp_v, tp_v, hh_v, rh_v, th_v, h_v, r_v, t_v, o_v,
          sem):
        wid = lax.axis_index("s") * _NUM_CORES + lax.axis_index("c")
        row0 = wid * rows_pw
        tri_iota = lax.broadcasted_iota(jnp.int32, (_LANES,), 0)

        pltpu.sync_copy(hp_hbm.at[pl.ds(row0, rows_pw)], hp_v)
        pltpu.sync_copy(rp_hbm.at[pl.ds(row0, rows_pw)], rp_v)
        pltpu.sync_copy(tp_hbm.at[pl.ds(row0, rows_pw)], tp_v)
        pltpu.sync_copy(hh_hbm.at[pl.ds(row0, rows_pw)], hh_v)
        pltpu.sync_copy(rh_hbm.at[pl.ds(row0, rows_pw)], rh_v)
        pltpu.sync_copy(th_hbm.at[pl.ds(row0, rows_pw)], th_v)

        @pl.loop(0, n_half)
        def _(half):
            irow0 = half * rows_ph
            copies = []
            for cr in range(rows_ph):
                dst = pl.ds(cr * _IDX_COLS, _IDX_COLS)
                copies.append(pltpu.async_copy(
                    ent_hbm.at[hp_v.at[irow0 + cr]], h_v.at[dst], sem))
                copies.append(pltpu.async_copy(
                    rel_hbm.at[rp_v.at[irow0 + cr]], r_v.at[dst], sem))
                copies.append(pltpu.async_copy(
                    ent_hbm.at[tp_v.at[irow0 + cr]], t_v.at[dst], sem))
            for cpy in copies:
                cpy.wait()

            @pl.loop(0, groups)
            def _(g):
                irow = irow0 + (g >> 3)
                icol0 = (g & 7) * _LANES
                rows16 = g * _LANES + tri_iota
                col_h = hh_v[irow, pl.ds(icol0, _LANES)]
                col_r = rh_v[irow, pl.ds(icol0, _LANES)]
                col_t = th_v[irow, pl.ds(icol0, _LANES)]
                acc = jnp.zeros((_LANES,), jnp.float32)
                for j in range(_HIDDEN):
                    hj = plsc.load_gather(h_v, [rows16, col_h + j])
                    rj = plsc.load_gather(r_v, [rows16, col_r + j])
                    tj = plsc.load_gather(t_v, [rows16, col_t + j])
                    acc = acc + jnp.abs(hj + rj - tj)
                o_v[pl.ds(half * _HALF + g * _LANES, _LANES)] = _GAMMA - acc

        pltpu.sync_copy(o_v, out_hbm.at[pl.ds(wid * bpw, bpw)])

    return k(ent_p, rel_p, hp_idx, rp_idx, tp_idx, hh_idx, rh_idx, th_idx)


def kernel(sample, entity_embedding, relation_embedding):
    batch = sample.shape[0]
    rows = batch // _IDX_COLS
    nent = entity_embedding.shape[0]
    dim = entity_embedding.shape[1]
    h = sample[:, 0]
    r = sample[:, 1]
    t = sample[:, 2]
    hp_idx = (h >> 1).reshape(rows, _IDX_COLS)
    rp_idx = (r >> 1).reshape(rows, _IDX_COLS)
    tp_idx = (t >> 1).reshape(rows, _IDX_COLS)
    hh_idx = ((h & 1) * dim).reshape(rows, _IDX_COLS)
    rh_idx = ((r & 1) * dim).reshape(rows, _IDX_COLS)
    th_idx = ((t & 1) * dim).reshape(rows, _IDX_COLS)
    ent_p = entity_embedding.reshape(nent // 2, 2 * dim)
    rel_p = relation_embedding.reshape(nent // 2, 2 * dim)
    score = _score(batch, hp_idx, rp_idx, tp_idx, hh_idx, rh_idx, th_idx,
                   ent_p, rel_p)
    return score.reshape(batch, 1)


# speedup vs baseline: 1.3290x; 1.3290x over previous
import dataclasses
import functools

import jax
import jax.numpy as jnp
from jax import lax
from jax.experimental import pallas as pl
from jax.experimental.pallas import tpu as pltpu
from jax.experimental.pallas import tpu_sc as plsc

_HIDDEN = 64
_GAMMA = 12.0
_LANES = 16
_NUM_CORES = 2
_NUM_SUBCORES = 16
_NUM_WORKERS = _NUM_CORES * _NUM_SUBCORES
_IDX_COLS = 128
_HALF = 256


@functools.partial(jax.jit, static_argnames=("batch",))
def _score(batch, hp_idx, rp_idx, tp_idx, hh_idx, rh_idx, th_idx,
           ent_p, rel_p):
    bpw = batch // _NUM_WORKERS
    rows_pw = bpw // _IDX_COLS
    n_half = bpw // _HALF
    rows_ph = _HALF // _IDX_COLS
    groups = _HALF // _LANES
    mesh = plsc.VectorSubcoreMesh(core_axis_name="c", subcore_axis_name="s")
    cp = pltpu.CompilerParams()
    if "needs_layout_passes" in pltpu.CompilerParams.__dataclass_fields__:
        cp = dataclasses.replace(cp, needs_layout_passes=False)

    @functools.partial(
        pl.kernel,
        out_type=jax.ShapeDtypeStruct((batch,), jnp.float32),
        mesh=mesh,
        compiler_params=cp,
        scratch_types=[
            pltpu.VMEM((rows_pw, _IDX_COLS), jnp.int32),
            pltpu.VMEM((rows_pw, _IDX_COLS), jnp.int32),
            pltpu.VMEM((rows_pw, _IDX_COLS), jnp.int32),
            pltpu.VMEM((rows_pw, _IDX_COLS), jnp.int32),
            pltpu.VMEM((rows_pw, _IDX_COLS), jnp.int32),
            pltpu.VMEM((rows_pw, _IDX_COLS), jnp.int32),
            pltpu.VMEM((_HALF, 2 * _HIDDEN), jnp.float32),
            pltpu.VMEM((_HALF, 2 * _HIDDEN), jnp.float32),
            pltpu.VMEM((_HALF, 2 * _HIDDEN), jnp.float32),
            pltpu.VMEM((bpw,), jnp.float32),
            pltpu.SemaphoreType.DMA,
        ],
    )
    def k(ent_hbm, rel_hbm, hp_hbm, rp_hbm, tp_hbm, hh_hbm, rh_hbm, th_hbm,
          out_hbm, hp_v, rp_v, tp_v, hh_v, rh_v, th_v, h_v, r_v, t_v, o_v,
          sem):
        wid = lax.axis_index("s") * _NUM_CORES + lax.axis_index("c")
        row0 = wid * rows_pw
        tri_iota = lax.broadcasted_iota(jnp.int32, (_LANES,), 0)

        pltpu.sync_copy(hp_hbm.at[pl.ds(row0, rows_pw)], hp_v)
        pltpu.sync_copy(rp_hbm.at[pl.ds(row0, rows_pw)], rp_v)
        pltpu.sync_copy(tp_hbm.at[pl.ds(row0, rows_pw)], tp_v)
        pltpu.sync_copy(hh_hbm.at[pl.ds(row0, rows_pw)], hh_v)
        pltpu.sync_copy(rh_hbm.at[pl.ds(row0, rows_pw)], rh_v)
        pltpu.sync_copy(th_hbm.at[pl.ds(row0, rows_pw)], th_v)

        @pl.loop(0, n_half)
        def _(half):
            irow0 = half * rows_ph
            copies = []
            for cr in range(rows_ph):
                dst = pl.ds(cr * _IDX_COLS, _IDX_COLS)
                copies.append(pltpu.async_copy(
                    ent_hbm.at[hp_v.at[irow0 + cr]], h_v.at[dst], sem))
                copies.append(pltpu.async_copy(
                    rel_hbm.at[rp_v.at[irow0 + cr]], r_v.at[dst], sem))
                copies.append(pltpu.async_copy(
                    ent_hbm.at[tp_v.at[irow0 + cr]], t_v.at[dst], sem))
            for cpy in copies:
                cpy.wait()

            @pl.loop(0, groups)
            def _(g):
                irow = irow0 + (g >> 3)
                icol0 = (g & 7) * _LANES
                rows16 = g * _LANES + tri_iota
                col_h = hh_v[irow, pl.ds(icol0, _LANES)]
                col_r = rh_v[irow, pl.ds(icol0, _LANES)]
                col_t = th_v[irow, pl.ds(icol0, _LANES)]
                acc = jnp.zeros((_LANES,), jnp.float32)
                for j in range(_HIDDEN):
                    hj = plsc.load_gather(h_v, [rows16, col_h + j])
                    rj = plsc.load_gather(r_v, [rows16, col_r + j])
                    tj = plsc.load_gather(t_v, [rows16, col_t + j])
                    acc = acc + jnp.abs(hj + rj - tj)
                o_v[pl.ds(half * _HALF + g * _LANES, _LANES)] = _GAMMA - acc

        pltpu.sync_copy(o_v, out_hbm.at[pl.ds(wid * bpw, bpw)])

    return k(ent_p, rel_p, hp_idx, rp_idx, tp_idx, hh_idx, rh_idx, th_idx)


def _repack_tc(table_t):
    dim, n = table_t.shape
    be = 2048
    split = 524288
    grid = split // be
    last = (n + be - 1) // be - 1

    def body(a_ref, b_ref, out_ref):
        out_ref[:, 0:dim] = jnp.swapaxes(a_ref[...], 0, 1)
        out_ref[:, dim:2 * dim] = jnp.swapaxes(b_ref[...], 0, 1)

    return pl.pallas_call(
        body,
        grid=(grid,),
        in_specs=[pl.BlockSpec((dim, be), lambda i: (0, i)),
                  pl.BlockSpec((dim, be),
                               lambda i: (0, jnp.minimum(i + grid, last)))],
        out_specs=pl.BlockSpec((be, 2 * dim), lambda i: (i, 0)),
        out_shape=jax.ShapeDtypeStruct((split, 2 * dim), jnp.float32),
    )(table_t, table_t)


def kernel(sample, entity_embedding, relation_embedding):
    batch = sample.shape[0]
    rows = batch // _IDX_COLS
    nent = entity_embedding.shape[0]
    dim = entity_embedding.shape[1]
    h = sample[:, 0]
    r = sample[:, 1]
    t = sample[:, 2]
    split = 524288
    hp_idx = jnp.where(h < split, h, h - split).reshape(rows, _IDX_COLS)
    tp_idx = jnp.where(t < split, t, t - split).reshape(rows, _IDX_COLS)
    hh_idx = jnp.where(h < split, 0, dim).reshape(rows, _IDX_COLS)
    th_idx = jnp.where(t < split, 0, dim).reshape(rows, _IDX_COLS)
    rp_idx = (r >> 1).reshape(rows, _IDX_COLS)
    rh_idx = ((r & 1) * dim).reshape(rows, _IDX_COLS)
    ent_p = _repack_tc(entity_embedding.T)
    rel_p = relation_embedding.reshape(nent // 2, 2 * dim)
    score = _score(batch, hp_idx, rp_idx, tp_idx, hh_idx, rh_idx, th_idx,
                   ent_p, rel_p)
    return score.reshape(batch, 1)


# speedup vs baseline: 1.5543x; 1.1695x over previous
import dataclasses
import functools

import jax
import jax.numpy as jnp
from jax import lax
from jax.experimental import pallas as pl
from jax.experimental.pallas import tpu as pltpu
from jax.experimental.pallas import tpu_sc as plsc

_HIDDEN = 64
_GAMMA = 12.0
_LANES = 16
_NUM_CORES = 2
_NUM_SUBCORES = 16
_NUM_WORKERS = _NUM_CORES * _NUM_SUBCORES
_IDX_COLS = 128
_HALF = 256


@functools.partial(jax.jit, static_argnames=("batch",))
def _score(batch, hp_idx, rp_idx, tp_idx, hh_idx, rh_idx, th_idx,
           ent_p, rel_p):
    bpw = batch // _NUM_WORKERS
    rows_pw = bpw // _IDX_COLS
    n_half = bpw // _HALF
    rows_ph = _HALF // _IDX_COLS
    groups = _HALF // _LANES
    mesh = plsc.VectorSubcoreMesh(core_axis_name="c", subcore_axis_name="s")
    cp = pltpu.CompilerParams()
    if "needs_layout_passes" in pltpu.CompilerParams.__dataclass_fields__:
        cp = dataclasses.replace(cp, needs_layout_passes=False)

    @functools.partial(
        pl.kernel,
        out_type=jax.ShapeDtypeStruct((batch,), jnp.float32),
        mesh=mesh,
        compiler_params=cp,
        scratch_types=[
            pltpu.VMEM((rows_pw, _IDX_COLS), jnp.int32),
            pltpu.VMEM((rows_pw, _IDX_COLS), jnp.int32),
            pltpu.VMEM((rows_pw, _IDX_COLS), jnp.int32),
            pltpu.VMEM((rows_pw, _IDX_COLS), jnp.int32),
            pltpu.VMEM((rows_pw, _IDX_COLS), jnp.int32),
            pltpu.VMEM((rows_pw, _IDX_COLS), jnp.int32),
            pltpu.VMEM((_HALF, 2 * _HIDDEN), jnp.float32),
            pltpu.VMEM((_HALF, 2 * _HIDDEN), jnp.float32),
            pltpu.VMEM((_HALF, 2 * _HIDDEN), jnp.float32),
            pltpu.VMEM((bpw,), jnp.float32),
            pltpu.SemaphoreType.DMA,
        ],
    )
    def k(ent_hbm, rel_hbm, hp_hbm, rp_hbm, tp_hbm, hh_hbm, rh_hbm, th_hbm,
          out_hbm, hp_v, rp_v, tp_v, hh_v, rh_v, th_v, h_v, r_v, t_v, o_v,
          sem):
        wid = lax.axis_index("s") * _NUM_CORES + lax.axis_index("c")
        row0 = wid * rows_pw
        tri_iota = lax.broadcasted_iota(jnp.int32, (_LANES,), 0)

        pltpu.sync_copy(hp_hbm.at[pl.ds(row0, rows_pw)], hp_v)
        pltpu.sync_copy(rp_hbm.at[pl.ds(row0, rows_pw)], rp_v)
        pltpu.sync_copy(tp_hbm.at[pl.ds(row0, rows_pw)], tp_v)
        pltpu.sync_copy(hh_hbm.at[pl.ds(row0, rows_pw)], hh_v)
        pltpu.sync_copy(rh_hbm.at[pl.ds(row0, rows_pw)], rh_v)
        pltpu.sync_copy(th_hbm.at[pl.ds(row0, rows_pw)], th_v)

        @pl.loop(0, n_half)
        def _(half):
            irow0 = half * rows_ph
            copies = []
            for cr in range(rows_ph):
                dst = pl.ds(cr * _IDX_COLS, _IDX_COLS)
                copies.append(pltpu.async_copy(
                    ent_hbm.at[hp_v.at[irow0 + cr]], h_v.at[dst], sem))
                copies.append(pltpu.async_copy(
                    rel_hbm.at[rp_v.at[irow0 + cr]], r_v.at[dst], sem))
                copies.append(pltpu.async_copy(
                    ent_hbm.at[tp_v.at[irow0 + cr]], t_v.at[dst], sem))
            for cpy in copies:
                cpy.wait()

            @pl.loop(0, groups)
            def _(g):
                irow = irow0 + (g >> 3)
                icol0 = (g & 7) * _LANES
                rows16 = g * _LANES + tri_iota
                col_h = hh_v[irow, pl.ds(icol0, _LANES)]
                col_r = rh_v[irow, pl.ds(icol0, _LANES)]
                col_t = th_v[irow, pl.ds(icol0, _LANES)]
                acc = jnp.zeros((_LANES,), jnp.float32)
                for j in range(_HIDDEN):
                    hj = plsc.load_gather(h_v, [rows16, col_h + j])
                    rj = plsc.load_gather(r_v, [rows16, col_r + j])
                    tj = plsc.load_gather(t_v, [rows16, col_t + j])
                    acc = acc + jnp.abs(hj + rj - tj)
                o_v[pl.ds(half * _HALF + g * _LANES, _LANES)] = _GAMMA - acc

        pltpu.sync_copy(o_v, out_hbm.at[pl.ds(wid * bpw, bpw)])

    return k(ent_p, rel_p, hp_idx, rp_idx, tp_idx, hh_idx, rh_idx, th_idx)


def _repack_tc(table_t):
    dim, n = table_t.shape
    be = 2048
    split = 524288
    grid = split // be
    last = (n + be - 1) // be - 1

    def body(a_ref, b_ref, out_ref):
        out_ref[:, 0:dim] = jnp.swapaxes(a_ref[...], 0, 1)
        out_ref[:, dim:2 * dim] = jnp.swapaxes(b_ref[...], 0, 1)

    return pl.pallas_call(
        body,
        grid=(grid,),
        in_specs=[pl.BlockSpec((dim, be), lambda i: (0, i)),
                  pl.BlockSpec((dim, be),
                               lambda i: (0, jnp.minimum(i + grid, last)))],
        out_specs=pl.BlockSpec((be, 2 * dim), lambda i: (i, 0)),
        out_shape=jax.ShapeDtypeStruct((split, 2 * dim), jnp.float32),
    )(table_t, table_t)


def kernel(sample, entity_embedding, relation_embedding):
    batch = sample.shape[0]
    rows = batch // _IDX_COLS
    nent = entity_embedding.shape[0]
    dim = entity_embedding.shape[1]
    h = sample[:, 0]
    r = sample[:, 1]
    t = sample[:, 2]
    split = 524288
    hp_idx = jnp.where(h < split, h, h - split).reshape(rows, _IDX_COLS)
    tp_idx = jnp.where(t < split, t, t - split).reshape(rows, _IDX_COLS)
    hh_idx = jnp.where(h < split, 0, dim).reshape(rows, _IDX_COLS)
    th_idx = jnp.where(t < split, 0, dim).reshape(rows, _IDX_COLS)
    rp_idx = jnp.where(r < split, r, r - split).reshape(rows, _IDX_COLS)
    rh_idx = jnp.where(r < split, 0, dim).reshape(rows, _IDX_COLS)
    ent_p = _repack_tc(entity_embedding.T)
    rel_p = _repack_tc(relation_embedding.T)
    score = _score(batch, hp_idx, rp_idx, tp_idx, hh_idx, rh_idx, th_idx,
                   ent_p, rel_p)
    return score.reshape(batch, 1)


# speedup vs baseline: 1.9392x; 1.2477x over previous
import dataclasses
import functools

import jax
import jax.numpy as jnp
from jax import lax
from jax.experimental import pallas as pl
from jax.experimental.pallas import tpu as pltpu
from jax.experimental.pallas import tpu_sc as plsc

_HIDDEN = 64
_GAMMA = 12.0
_LANES = 16
_NUM_CORES = 2
_NUM_SUBCORES = 16
_NUM_WORKERS = _NUM_CORES * _NUM_SUBCORES
_IDX_COLS = 128
_HALF = 256


@functools.partial(jax.jit, static_argnames=("batch",))
def _score(batch, hp_idx, rp_idx, tp_idx, hh_idx, rh_idx, th_idx,
           ent_p, rel_p):
    bpw = batch // _NUM_WORKERS
    rows_pw = bpw // _IDX_COLS
    n_half = bpw // _HALF
    rows_ph = _HALF // _IDX_COLS
    groups = _HALF // _LANES
    mesh = plsc.VectorSubcoreMesh(core_axis_name="c", subcore_axis_name="s")
    cp = pltpu.CompilerParams()
    if "needs_layout_passes" in pltpu.CompilerParams.__dataclass_fields__:
        cp = dataclasses.replace(cp, needs_layout_passes=False)

    @functools.partial(
        pl.kernel,
        out_type=jax.ShapeDtypeStruct((batch,), jnp.float32),
        mesh=mesh,
        compiler_params=cp,
        scratch_types=[
            pltpu.VMEM((rows_pw, _IDX_COLS), jnp.int32),
            pltpu.VMEM((rows_pw, _IDX_COLS), jnp.int32),
            pltpu.VMEM((rows_pw, _IDX_COLS), jnp.int32),
            pltpu.VMEM((rows_pw, _IDX_COLS), jnp.int32),
            pltpu.VMEM((rows_pw, _IDX_COLS), jnp.int32),
            pltpu.VMEM((rows_pw, _IDX_COLS), jnp.int32),
            pltpu.VMEM((_HALF, 2 * _HIDDEN), jnp.float32),
            pltpu.VMEM((_HALF, 2 * _HIDDEN), jnp.float32),
            pltpu.VMEM((_HALF, 2 * _HIDDEN), jnp.float32),
            pltpu.VMEM((bpw,), jnp.float32),
            pltpu.SemaphoreType.DMA,
        ],
    )
    def k(ent_hbm, rel_hbm, hp_hbm, rp_hbm, tp_hbm, hh_hbm, rh_hbm, th_hbm,
          out_hbm, hp_v, rp_v, tp_v, hh_v, rh_v, th_v, h_v, r_v, t_v, o_v,
          sem):
        wid = lax.axis_index("s") * _NUM_CORES + lax.axis_index("c")
        row0 = wid * rows_pw
        tri_iota = lax.broadcasted_iota(jnp.int32, (_LANES,), 0)

        pltpu.sync_copy(hp_hbm.at[pl.ds(row0, rows_pw)], hp_v)
        pltpu.sync_copy(rp_hbm.at[pl.ds(row0, rows_pw)], rp_v)
        pltpu.sync_copy(tp_hbm.at[pl.ds(row0, rows_pw)], tp_v)
        pltpu.sync_copy(hh_hbm.at[pl.ds(row0, rows_pw)], hh_v)
        pltpu.sync_copy(rh_hbm.at[pl.ds(row0, rows_pw)], rh_v)
        pltpu.sync_copy(th_hbm.at[pl.ds(row0, rows_pw)], th_v)

        @pl.loop(0, n_half)
        def _(half):
            irow0 = half * rows_ph
            copies = []
            for cr in range(rows_ph):
                dst = pl.ds(cr * _IDX_COLS, _IDX_COLS)
                copies.append(pltpu.async_copy(
                    ent_hbm.at[hp_v.at[irow0 + cr]], h_v.at[dst], sem))
                copies.append(pltpu.async_copy(
                    rel_hbm.at[rp_v.at[irow0 + cr]], r_v.at[dst], sem))
                copies.append(pltpu.async_copy(
                    ent_hbm.at[tp_v.at[irow0 + cr]], t_v.at[dst], sem))
            for cpy in copies:
                cpy.wait()

            @pl.loop(0, groups)
            def _(g):
                irow = irow0 + (g >> 3)
                icol0 = (g & 7) * _LANES
                rows16 = g * _LANES + tri_iota
                col_h = hh_v[irow, pl.ds(icol0, _LANES)]
                col_r = rh_v[irow, pl.ds(icol0, _LANES)]
                col_t = th_v[irow, pl.ds(icol0, _LANES)]
                acc = jnp.zeros((_LANES,), jnp.float32)
                for j in range(_HIDDEN):
                    hj = plsc.load_gather(h_v, [rows16, col_h + j])
                    rj = plsc.load_gather(r_v, [rows16, col_r + j])
                    tj = plsc.load_gather(t_v, [rows16, col_t + j])
                    acc = acc + jnp.abs(hj + rj - tj)
                o_v[pl.ds(half * _HALF + g * _LANES, _LANES)] = _GAMMA - acc

        pltpu.sync_copy(o_v, out_hbm.at[pl.ds(wid * bpw, bpw)])

    return k(ent_p, rel_p, hp_idx, rp_idx, tp_idx, hh_idx, rh_idx, th_idx)


def _repack_tc(ent_t, rel_t):
    dim, n = ent_t.shape
    be = 2048
    split = 524288
    grid = split // be
    last = (n + be - 1) // be - 1

    def body(ea_ref, eb_ref, ra_ref, rb_ref, eout_ref, rout_ref):
        eout_ref[:, 0:dim] = jnp.swapaxes(ea_ref[...], 0, 1)
        rout_ref[:, 0:dim] = jnp.swapaxes(ra_ref[...], 0, 1)
        eout_ref[:, dim:2 * dim] = jnp.swapaxes(eb_ref[...], 0, 1)
        rout_ref[:, dim:2 * dim] = jnp.swapaxes(rb_ref[...], 0, 1)

    lo_spec = pl.BlockSpec((dim, be), lambda i: (0, i))
    hi_spec = pl.BlockSpec((dim, be),
                           lambda i: (0, jnp.minimum(i + grid, last)))
    out_spec = pl.BlockSpec((be, 2 * dim), lambda i: (i, 0))
    out_sds = jax.ShapeDtypeStruct((split, 2 * dim), jnp.float32)
    return pl.pallas_call(
        body,
        grid=(grid,),
        in_specs=[lo_spec, hi_spec, lo_spec, hi_spec],
        out_specs=[out_spec, out_spec],
        out_shape=[out_sds, out_sds],
    )(ent_t, ent_t, rel_t, rel_t)


def kernel(sample, entity_embedding, relation_embedding):
    batch = sample.shape[0]
    rows = batch // _IDX_COLS
    nent = entity_embedding.shape[0]
    dim = entity_embedding.shape[1]
    h = sample[:, 0]
    r = sample[:, 1]
    t = sample[:, 2]
    split = 524288
    hp_idx = jnp.where(h < split, h, h - split).reshape(rows, _IDX_COLS)
    tp_idx = jnp.where(t < split, t, t - split).reshape(rows, _IDX_COLS)
    hh_idx = jnp.where(h < split, 0, dim).reshape(rows, _IDX_COLS)
    th_idx = jnp.where(t < split, 0, dim).reshape(rows, _IDX_COLS)
    rp_idx = jnp.where(r < split, r, r - split).reshape(rows, _IDX_COLS)
    rh_idx = jnp.where(r < split, 0, dim).reshape(rows, _IDX_COLS)
    ent_p, rel_p = _repack_tc(entity_embedding.T, relation_embedding.T)
    score = _score(batch, hp_idx, rp_idx, tp_idx, hh_idx, rh_idx, th_idx,
                   ent_p, rel_p)
    return score.reshape(batch, 1)


# speedup vs baseline: 1.9456x; 1.0033x over previous
import dataclasses
import functools

import jax
import jax.numpy as jnp
from jax import lax
from jax.experimental import pallas as pl
from jax.experimental.pallas import tpu as pltpu
from jax.experimental.pallas import tpu_sc as plsc

_HIDDEN = 64
_GAMMA = 12.0
_LANES = 16
_NUM_CORES = 2
_NUM_SUBCORES = 16
_NUM_WORKERS = _NUM_CORES * _NUM_SUBCORES
_IDX_COLS = 128
_HALF = 256


@functools.partial(jax.jit, static_argnames=("batch",))
def _score(batch, sample_t, ent_p, rel_p):
    split = ent_p.shape[0]
    bpw = batch // _NUM_WORKERS
    rows_pw = bpw // _IDX_COLS
    n_half = bpw // _HALF
    rows_ph = _HALF // _IDX_COLS
    groups = _HALF // _LANES
    mesh = plsc.VectorSubcoreMesh(core_axis_name="c", subcore_axis_name="s")
    cp = pltpu.CompilerParams()
    if "needs_layout_passes" in pltpu.CompilerParams.__dataclass_fields__:
        cp = dataclasses.replace(cp, needs_layout_passes=False)

    @functools.partial(
        pl.kernel,
        out_type=jax.ShapeDtypeStruct((batch,), jnp.float32),
        mesh=mesh,
        compiler_params=cp,
        scratch_types=[
            pltpu.VMEM((3, bpw), jnp.int32),
            pltpu.VMEM((rows_pw, _IDX_COLS), jnp.int32),
            pltpu.VMEM((rows_pw, _IDX_COLS), jnp.int32),
            pltpu.VMEM((rows_pw, _IDX_COLS), jnp.int32),
            pltpu.VMEM((rows_pw, _IDX_COLS), jnp.int32),
            pltpu.VMEM((rows_pw, _IDX_COLS), jnp.int32),
            pltpu.VMEM((rows_pw, _IDX_COLS), jnp.int32),
            pltpu.VMEM((_HALF, 2 * _HIDDEN), jnp.float32),
            pltpu.VMEM((_HALF, 2 * _HIDDEN), jnp.float32),
            pltpu.VMEM((_HALF, 2 * _HIDDEN), jnp.float32),
            pltpu.VMEM((bpw,), jnp.float32),
            pltpu.SemaphoreType.DMA,
        ],
    )
    def k(ent_hbm, rel_hbm, s_hbm, out_hbm,
          s_v, hp_v, rp_v, tp_v, hh_v, rh_v, th_v, h_v, r_v, t_v, o_v,
          sem):
        wid = lax.axis_index("s") * _NUM_CORES + lax.axis_index("c")
        tri_iota = lax.broadcasted_iota(jnp.int32, (_LANES,), 0)

        pltpu.sync_copy(s_hbm.at[:, pl.ds(wid * bpw, bpw)], s_v)

        @pl.loop(0, rows_pw * (_IDX_COLS // _LANES))
        def _(m):
            irow = m >> 3
            icol = (m & 7) * _LANES
            sl = pl.ds(m * _LANES, _LANES)
            dsl = pl.ds(icol, _LANES)
            hv = s_v[0, sl]
            rv = s_v[1, sl]
            tv = s_v[2, sl]
            hp_v[irow, dsl] = jnp.where(hv < split, hv, hv - split)
            hh_v[irow, dsl] = jnp.where(hv < split, 0, _HIDDEN)
            rp_v[irow, dsl] = jnp.where(rv < split, rv, rv - split)
            rh_v[irow, dsl] = jnp.where(rv < split, 0, _HIDDEN)
            tp_v[irow, dsl] = jnp.where(tv < split, tv, tv - split)
            th_v[irow, dsl] = jnp.where(tv < split, 0, _HIDDEN)

        @pl.loop(0, n_half)
        def _(half):
            irow0 = half * rows_ph
            copies = []
            for cr in range(rows_ph):
                dst = pl.ds(cr * _IDX_COLS, _IDX_COLS)
                copies.append(pltpu.async_copy(
                    ent_hbm.at[hp_v.at[irow0 + cr]], h_v.at[dst], sem))
                copies.append(pltpu.async_copy(
                    rel_hbm.at[rp_v.at[irow0 + cr]], r_v.at[dst], sem))
                copies.append(pltpu.async_copy(
                    ent_hbm.at[tp_v.at[irow0 + cr]], t_v.at[dst], sem))
            for cpy in copies:
                cpy.wait()

            @pl.loop(0, groups)
            def _(g):
                irow = irow0 + (g >> 3)
                icol0 = (g & 7) * _LANES
                rows16 = g * _LANES + tri_iota
                col_h = hh_v[irow, pl.ds(icol0, _LANES)]
                col_r = rh_v[irow, pl.ds(icol0, _LANES)]
                col_t = th_v[irow, pl.ds(icol0, _LANES)]
                accs = [jnp.zeros((_LANES,), jnp.float32) for _ in range(4)]
                for j in range(_HIDDEN):
                    hj = plsc.load_gather(h_v, [rows16, col_h + j])
                    rj = plsc.load_gather(r_v, [rows16, col_r + j])
                    tj = plsc.load_gather(t_v, [rows16, col_t + j])
                    accs[j & 3] = accs[j & 3] + jnp.abs(hj + rj - tj)
                acc = (accs[0] + accs[1]) + (accs[2] + accs[3])
                o_v[pl.ds(half * _HALF + g * _LANES, _LANES)] = _GAMMA - acc

        pltpu.sync_copy(o_v, out_hbm.at[pl.ds(wid * bpw, bpw)])

    return k(ent_p, rel_p, sample_t)


def _repack_tc(ent_t, rel_t):
    dim, n = ent_t.shape
    be = 2048
    split = 524288
    grid = split // be
    last = (n + be - 1) // be - 1

    def body(ea_ref, eb_ref, ra_ref, rb_ref, eout_ref, rout_ref):
        eout_ref[:, 0:dim] = jnp.swapaxes(ea_ref[...], 0, 1)
        rout_ref[:, 0:dim] = jnp.swapaxes(ra_ref[...], 0, 1)
        eout_ref[:, dim:2 * dim] = jnp.swapaxes(eb_ref[...], 0, 1)
        rout_ref[:, dim:2 * dim] = jnp.swapaxes(rb_ref[...], 0, 1)

    lo_spec = pl.BlockSpec((dim, be), lambda i: (0, i))
    hi_spec = pl.BlockSpec((dim, be),
                           lambda i: (0, jnp.minimum(i + grid, last)))
    out_spec = pl.BlockSpec((be, 2 * dim), lambda i: (i, 0))
    out_sds = jax.ShapeDtypeStruct((split, 2 * dim), jnp.float32)
    return pl.pallas_call(
        body,
        grid=(grid,),
        in_specs=[lo_spec, hi_spec, lo_spec, hi_spec],
        out_specs=[out_spec, out_spec],
        out_shape=[out_sds, out_sds],
    )(ent_t, ent_t, rel_t, rel_t)


def kernel(sample, entity_embedding, relation_embedding):
    batch = sample.shape[0]
    ent_p, rel_p = _repack_tc(entity_embedding.T, relation_embedding.T)
    score = _score(batch, sample.T, ent_p, rel_p)
    return score.reshape(batch, 1)
